# Initial kernel scaffold; baseline (speedup 1.0000x reference)
import jax
import jax.numpy as jnp
from jax.experimental import pallas as pl


def kernel(det_boxes, det_scores, det_labels, boxes, labels):
    raise NotImplementedError("write your pallas kernel here")


# speedup vs baseline: 5.4622x; 5.4622x over previous
import jax
import jax.numpy as jnp
from jax.experimental import pallas as pl

_N = 5000
_NPAD = 5120
_NOBJ = 64
_THRESH = 0.5
_NEG_POS_RATIO = 3.0
_ALPHA = 1.0
_BISECT_ITERS = 40


def _attack_loss_body(dbt_ref, ds_ref, b_ref, lab_ref, out_ref):
    dbt = dbt_ref[...]
    ds = ds_ref[...]
    b = b_ref[...]
    lab = lab_ref[...]

    dx1 = dbt[0:1, :]
    dy1 = dbt[1:2, :]
    dx2 = dbt[2:3, :]
    dy2 = dbt[3:4, :]
    bx1 = b[:, 0:1]
    by1 = b[:, 1:2]
    bx2 = b[:, 2:3]
    by2 = b[:, 3:4]

    iw = jnp.maximum(jnp.minimum(bx2, dx2) - jnp.maximum(bx1, dx1), 0.0)
    ih = jnp.maximum(jnp.minimum(by2, dy2) - jnp.maximum(by1, dy1), 0.0)
    inter = iw * ih
    a1 = (bx2 - bx1) * (by2 - by1)
    a2 = (dx2 - dx1) * (dy2 - dy1)
    ov = inter / (a1 + a2 - inter)

    lane = jax.lax.broadcasted_iota(jnp.int32, (1, _NPAD), 1)
    valid = lane < _N
    oid = jax.lax.broadcasted_iota(jnp.int32, (_NOBJ, _NPAD), 0)
    lane2 = jax.lax.broadcasted_iota(jnp.int32, (_NOBJ, _NPAD), 1)

    ov_det0 = jnp.max(ov, axis=0, keepdims=True)
    obj_det0 = jnp.min(jnp.where(ov == ov_det0, oid, _NOBJ),
                       axis=0, keepdims=True)

    ovm = jnp.where(valid, ov, -3.0e38)
    row_max = jnp.max(ovm, axis=1, keepdims=True)
    det_obj = jnp.min(jnp.where(ovm == row_max, lane2, _NPAD),
                      axis=1, keepdims=True)

    hit = det_obj == lane
    ovr = jnp.max(jnp.where(hit, oid, -1), axis=0, keepdims=True)
    has_ovr = ovr >= 0
    final_obj = jnp.where(has_ovr, ovr, obj_det0)
    final_ov = jnp.where(has_ovr, 1.0, ov_det0)

    onehot = oid == final_obj
    lab_det = jnp.sum(jnp.where(onehot, lab, 0.0), axis=0, keepdims=True)
    keep = jnp.logical_not(final_ov < _THRESH)
    pos = (lab_det != 0.0) & keep & valid
    posf = jnp.where(pos, 1.0, 0.0)
    n_pos = jnp.sum(posf)

    l1 = jnp.zeros((1, _NPAD), jnp.float32)
    for c in range(4):
        tl = jnp.sum(jnp.where(onehot, b[:, c:c + 1], 0.0), axis=0,
                     keepdims=True)
        l1 = l1 + jnp.abs(dbt[c:c + 1, :] - tl)
    loc_loss = jnp.sum(l1 * posf) / (n_pos * 4.0)

    conf_all = 1.0 - ds
    conf_pos_sum = jnp.sum(conf_all * posf)
    conf_neg = jnp.where(pos | jnp.logical_not(valid), 0.0, conf_all)

    k_c = jnp.minimum(_NEG_POS_RATIO * n_pos, float(_N))

    def bis_body(_, lohi):
        lo, hi = lohi
        mid = 0.5 * (lo + hi)
        cg = jnp.sum(jnp.where(conf_neg > mid, 1.0, 0.0))
        p = cg < k_c
        return jnp.where(p, lo, mid), jnp.where(p, mid, hi)

    _, t = jax.lax.fori_loop(0, _BISECT_ITERS, bis_body,
                             (jnp.float32(0.0), jnp.float32(1.0)))
    gt = conf_neg > t
    cg_t = jnp.sum(jnp.where(gt, 1.0, 0.0))
    conf_hard_sum = jnp.sum(jnp.where(gt, conf_neg, 0.0)) + (k_c - cg_t) * t

    conf_loss = (conf_hard_sum + conf_pos_sum) / n_pos
    out_ref[...] = jnp.broadcast_to(conf_loss + _ALPHA * loc_loss, (1, 1))


@jax.jit
def kernel(det_boxes, det_scores, det_labels, boxes, labels):
    del det_labels
    db = det_boxes[0].astype(jnp.float32)
    ds = det_scores[0].astype(jnp.float32)
    b = boxes[0].astype(jnp.float32)
    lab = labels[0].astype(jnp.float32).reshape(_NOBJ, 1)

    dbt = jnp.pad(db.T, ((0, 0), (0, _NPAD - _N)))
    dsp = jnp.pad(ds.reshape(1, _N), ((0, 0), (0, _NPAD - _N)),
                  constant_values=1.0)

    out = pl.pallas_call(
        _attack_loss_body,
        out_shape=jax.ShapeDtypeStruct((1, 1), jnp.float32),
    )(dbt, dsp, b, lab)
    return out[0, 0]


# speedup vs baseline: 6.0933x; 1.1155x over previous
import jax
import jax.numpy as jnp
from jax.experimental import pallas as pl

_N = 5000
_NOBJ = 64
_THRESH = 0.5
_NEG_POS_RATIO = 3.0
_ALPHA = 1.0
_BISECT_ITERS = 26


def _attack_loss_body(db_ref, ds_ref, b_ref, lab_ref, out_ref):
    db = db_ref[0]
    ds = ds_ref[...]
    b = b_ref[0]
    lab = jnp.transpose(lab_ref[...]).astype(jnp.float32)
    dbt = jnp.transpose(db)

    dx1 = dbt[0:1, :]
    dy1 = dbt[1:2, :]
    dx2 = dbt[2:3, :]
    dy2 = dbt[3:4, :]
    bx1 = b[:, 0:1]
    by1 = b[:, 1:2]
    bx2 = b[:, 2:3]
    by2 = b[:, 3:4]

    iw = jnp.maximum(jnp.minimum(bx2, dx2) - jnp.maximum(bx1, dx1), 0.0)
    ih = jnp.maximum(jnp.minimum(by2, dy2) - jnp.maximum(by1, dy1), 0.0)
    inter = iw * ih
    a1 = (bx2 - bx1) * (by2 - by1)
    a2 = (dx2 - dx1) * (dy2 - dy1)
    ov = inter / (a1 + a2 - inter)

    lane = jax.lax.broadcasted_iota(jnp.int32, (1, _N), 1)
    oid = jax.lax.broadcasted_iota(jnp.int32, (_NOBJ, _N), 0)
    lane2 = jax.lax.broadcasted_iota(jnp.int32, (_NOBJ, _N), 1)

    ov_det0 = jnp.max(ov, axis=0, keepdims=True)
    obj_det0 = jnp.min(jnp.where(ov == ov_det0, oid, _NOBJ),
                       axis=0, keepdims=True)

    row_max = jnp.max(ov, axis=1, keepdims=True)
    det_obj = jnp.min(jnp.where(ov == row_max, lane2, _N),
                      axis=1, keepdims=True)

    hit = det_obj == lane
    ovr = jnp.max(jnp.where(hit, oid, -1), axis=0, keepdims=True)
    has_ovr = ovr >= 0
    final_obj = jnp.where(has_ovr, ovr, obj_det0)
    final_ov = jnp.where(has_ovr, 1.0, ov_det0)

    onehot = oid == final_obj
    lab_det = jnp.sum(jnp.where(onehot, lab, 0.0), axis=0, keepdims=True)
    keep = jnp.logical_not(final_ov < _THRESH)
    pos = (lab_det != 0.0) & keep
    posf = jnp.where(pos, 1.0, 0.0)
    n_pos = jnp.sum(posf)

    l1 = jnp.zeros((1, _N), jnp.float32)
    for c in range(4):
        tl = jnp.sum(jnp.where(onehot, b[:, c:c + 1], 0.0), axis=0,
                     keepdims=True)
        l1 = l1 + jnp.abs(dbt[c:c + 1, :] - tl)
    loc_loss = jnp.sum(l1 * posf) / (n_pos * 4.0)

    conf_all = 1.0 - ds
    conf_pos_sum = jnp.sum(conf_all * posf)
    conf_neg = jnp.where(pos, 0.0, conf_all)

    k_c = jnp.minimum(_NEG_POS_RATIO * n_pos, float(_N))

    def bis_body(_, lohi):
        lo, hi = lohi
        mid = 0.5 * (lo + hi)
        cg = jnp.sum(jnp.where(conf_neg > mid, 1.0, 0.0))
        p = cg < k_c
        return jnp.where(p, lo, mid), jnp.where(p, mid, hi)

    _, t = jax.lax.fori_loop(0, _BISECT_ITERS, bis_body,
                             (jnp.float32(0.0), jnp.float32(1.0)))
    gt = conf_neg > t
    cg_t = jnp.sum(jnp.where(gt, 1.0, 0.0))
    conf_hard_sum = jnp.sum(jnp.where(gt, conf_neg, 0.0)) + (k_c - cg_t) * t

    conf_loss = (conf_hard_sum + conf_pos_sum) / n_pos
    out_ref[...] = jnp.broadcast_to(conf_loss + _ALPHA * loc_loss, (1, 1))


@jax.jit
def kernel(det_boxes, det_scores, det_labels, boxes, labels):
    del det_labels
    out = pl.pallas_call(
        _attack_loss_body,
        out_shape=jax.ShapeDtypeStruct((1, 1), jnp.float32),
    )(det_boxes.astype(jnp.float32), det_scores.astype(jnp.float32),
      boxes.astype(jnp.float32), labels.astype(jnp.int32))
    return out[0, 0]


# speedup vs baseline: 7.2016x; 1.1819x over previous
import jax
import jax.numpy as jnp
from jax.experimental import pallas as pl

_N = 5000
_NOBJ = 64
_THRESH = 0.5
_NEG_POS_RATIO = 3.0
_ALPHA = 1.0
_SEARCH_ROUNDS = 9


def _attack_loss_body(db_ref, ds_ref, b_ref, lab_ref, out_ref):
    db = db_ref[0]
    ds = ds_ref[...]
    b = b_ref[0]
    lab_row = lab_ref[...].astype(jnp.float32)
    dbt = jnp.transpose(db)

    dx1 = dbt[0:1, :]
    dy1 = dbt[1:2, :]
    dx2 = dbt[2:3, :]
    dy2 = dbt[3:4, :]
    bx1 = b[:, 0:1]
    by1 = b[:, 1:2]
    bx2 = b[:, 2:3]
    by2 = b[:, 3:4]

    iw = jnp.maximum(jnp.minimum(bx2, dx2) - jnp.maximum(bx1, dx1), 0.0)
    ih = jnp.maximum(jnp.minimum(by2, dy2) - jnp.maximum(by1, dy1), 0.0)
    inter = iw * ih
    a1 = (bx2 - bx1) * (by2 - by1)
    a2 = (dx2 - dx1) * (dy2 - dy1)
    ov = inter / (a1 + a2 - inter)

    lane = jax.lax.broadcasted_iota(jnp.int32, (1, _N), 1)
    oid = jax.lax.broadcasted_iota(jnp.int32, (_NOBJ, _N), 0)
    lane2 = jax.lax.broadcasted_iota(jnp.int32, (_NOBJ, _N), 1)

    ov_det0 = jnp.max(ov, axis=0, keepdims=True)
    obj_det0 = jnp.min(jnp.where(ov == ov_det0, oid, _NOBJ),
                       axis=0, keepdims=True)

    row_max = jnp.max(ov, axis=1, keepdims=True)
    det_obj = jnp.min(jnp.where(ov == row_max, lane2, _N),
                      axis=1, keepdims=True)

    hit = det_obj == lane
    ovr = jnp.max(jnp.where(hit, oid, -1), axis=0, keepdims=True)
    has_ovr = ovr >= 0
    final_obj = jnp.where(has_ovr, ovr, obj_det0)
    final_ov = jnp.where(has_ovr, 1.0, ov_det0)

    onehot_f = jnp.where(oid == final_obj, 1.0, 0.0)
    lhs = jnp.concatenate([jnp.transpose(b), lab_row], axis=0)
    sel = jax.lax.dot_general(lhs, onehot_f, (((1,), (0,)), ((), ())),
                              preferred_element_type=jnp.float32)
    lab_det = sel[4:5, :]
    keep = jnp.logical_not(final_ov < _THRESH)
    pos = (lab_det != 0.0) & keep
    posf = jnp.where(pos, 1.0, 0.0)
    n_pos = jnp.sum(posf)

    l1 = jnp.zeros((1, _N), jnp.float32)
    for c in range(4):
        l1 = l1 + jnp.abs(dbt[c:c + 1, :] - sel[c:c + 1, :])
    loc_loss = jnp.sum(l1 * posf) / (n_pos * 4.0)

    conf_all = 1.0 - ds
    conf_pos_sum = jnp.sum(conf_all * posf)
    conf_neg = jnp.where(pos, 0.0, conf_all)

    k_c = jnp.minimum(_NEG_POS_RATIO * n_pos, float(_N))

    def search_round(_, low):
        lo, w = low
        wh = w * 0.125
        s = jnp.float32(0.0)
        for j in range(1, 8):
            probe = lo + wh * float(j)
            cg = jnp.sum(jnp.where(conf_neg > probe, 1.0, 0.0))
            s = s + jnp.where(cg >= k_c, 1.0, 0.0)
        return lo + s * wh, wh

    lo, w = jax.lax.fori_loop(0, _SEARCH_ROUNDS, search_round,
                              (jnp.float32(0.0), jnp.float32(1.0)))
    t = lo + w
    gt = conf_neg > t
    cg_t = jnp.sum(jnp.where(gt, 1.0, 0.0))
    conf_hard_sum = jnp.sum(jnp.where(gt, conf_neg, 0.0)) + (k_c - cg_t) * t

    conf_loss = (conf_hard_sum + conf_pos_sum) / n_pos
    out_ref[...] = jnp.broadcast_to(conf_loss + _ALPHA * loc_loss, (1, 1))


@jax.jit
def kernel(det_boxes, det_scores, det_labels, boxes, labels):
    del det_labels
    out = pl.pallas_call(
        _attack_loss_body,
        out_shape=jax.ShapeDtypeStruct((1, 1), jnp.float32),
    )(det_boxes.astype(jnp.float32), det_scores.astype(jnp.float32),
      boxes.astype(jnp.float32), labels.astype(jnp.int32))
    return out[0, 0]
